# baseline (speedup 1.0000x reference)
import jax
import jax.numpy as jnp
from jax import lax
from jax.experimental import pallas as pl
from jax.experimental.pallas import tpu as pltpu
from jax.experimental.pallas import tpu_sc as plsc

_NEG = 0.01
_EPS = 1e-5
_N = 10000
_E = 320000
_DE = 128
_H = 64

_NW = 32
_EPW = _E // _NW
_CH = 80
_NCH = _EPW // _CH

_TILE = 2560
_GRID = _E // _TILE


def _hdot(a, b):
    return jnp.dot(a, b, preferred_element_type=jnp.float32,
                   precision=lax.Precision.HIGHEST)


def _leaky(h):
    return jnp.where(h > 0, h, _NEG * h)


def _pad8(v):
    return jnp.broadcast_to(v[None, :], (8, v.shape[0]))


def _sc_gather_body(y_hbm, row_hbm, col_hbm, w_hbm, st_hbm,
                    idx_r, idx_c, bufr, bufc, stbuf,
                    sem_r, sem_c, sem_w):
    wid = lax.axis_index("s") * 2 + lax.axis_index("c")
    ebase = wid * _EPW

    def issue(k, b):
        eoff = ebase + k * _CH
        pltpu.sync_copy(row_hbm.at[pl.ds(eoff, _CH)], idx_r.at[b])
        pltpu.sync_copy(col_hbm.at[pl.ds(eoff, _CH)], idx_c.at[b])
        pltpu.async_copy(y_hbm.at[idx_r.at[b]], bufr.at[b], sem_r)
        pltpu.async_copy(y_hbm.at[idx_c.at[b]], bufc.at[b], sem_c)

    def drain(b):
        pltpu.make_async_copy(y_hbm.at[idx_r.at[b]], bufr.at[b], sem_r).wait()
        pltpu.make_async_copy(y_hbm.at[idx_c.at[b]], bufc.at[b], sem_c).wait()

    def accum(b, acc):
        for i in range(_CH):
            for f in range(4):
                vr = bufr[b, i, pl.ds(f * 16, 16)]
                vc = bufc[b, i, pl.ds(f * 16, 16)]
                bufr[b, i, pl.ds(_H + f * 16, 16)] = vc
                acc[f] = acc[f] + vr
                acc[4 + f] = acc[4 + f] + vr * vr
                acc[8 + f] = acc[8 + f] + vc
                acc[12 + f] = acc[12 + f] + vc * vc
        return acc

    issue(0, 0)

    def chunk2(j, carry):
        acc = list(carry)
        for b in range(2):
            k = 2 * j + b
            drain(b)
            @pl.when(k > 0)
            def _():
                pltpu.make_async_copy(
                    bufr.at[1 - b], w_hbm.at[pl.ds(0, _CH)], sem_w).wait()
            issue(k + 1, 1 - b)
            acc = accum(b, acc)
            pltpu.async_copy(
                bufr.at[b], w_hbm.at[pl.ds(ebase + k * _CH, _CH)], sem_w)
        return tuple(acc)

    zero = jnp.zeros((16,), jnp.float32)
    acc = lax.fori_loop(0, _NCH // 2, chunk2, tuple(zero for _ in range(16)))
    k = _NCH - 1
    drain(0)
    pltpu.make_async_copy(
        bufr.at[1], w_hbm.at[pl.ds(0, _CH)], sem_w).wait()
    acc = accum(0, list(acc))
    pltpu.async_copy(bufr.at[0], w_hbm.at[pl.ds(ebase + k * _CH, _CH)], sem_w)
    pltpu.make_async_copy(bufr.at[0], w_hbm.at[pl.ds(0, _CH)], sem_w).wait()
    for r in range(4):
        for f in range(4):
            stbuf[r, pl.ds(f * 16, 16)] = acc[r * 4 + f]
    pltpu.sync_copy(stbuf, st_hbm.at[wid])


def _sc_gather(y, row, col):
    fn = pl.kernel(
        _sc_gather_body,
        out_type=[
            jax.ShapeDtypeStruct((_E, 2 * _H), jnp.float32),
            jax.ShapeDtypeStruct((_NW, 4, _H), jnp.float32),
        ],
        scratch_types=[
            pltpu.VMEM((2, _CH), jnp.int32),
            pltpu.VMEM((2, _CH), jnp.int32),
            pltpu.VMEM((2, _CH, 2 * _H), jnp.float32),
            pltpu.VMEM((2, _CH, 2 * _H), jnp.float32),
            pltpu.VMEM((4, _H), jnp.float32),
            pltpu.SemaphoreType.DMA,
            pltpu.SemaphoreType.DMA,
            pltpu.SemaphoreType.DMA,
        ],
        mesh=plsc.VectorSubcoreMesh(core_axis_name="c", subcore_axis_name="s"),
    )
    return fn(y, row, col)


def _node_y_body(x_ref, w_ref, b_ref, o_ref):
    h = jnp.dot(x_ref[...], w_ref[...], preferred_element_type=jnp.float32, precision=lax.Precision.HIGHEST)
    y = _leaky(h + b_ref[0:1, :])
    o_ref[...] = jnp.concatenate(
        [y, jnp.zeros((_N, _H), jnp.float32)], axis=1)


def _node_y(x, w, b8):
    return pl.pallas_call(
        _node_y_body,
        out_shape=jax.ShapeDtypeStruct((_N, 2 * _H), jnp.float32),
    )(x, w, b8)


def _accum_stats(st_ref, h):
    @pl.when(pl.program_id(0) == 0)
    def _():
        st_ref[...] = jnp.zeros_like(st_ref)

    s = jnp.sum(h, axis=0, keepdims=True)
    q = jnp.sum(h * h, axis=0, keepdims=True)
    st_ref[...] += jnp.concatenate(
        [s, q, jnp.zeros((6, _H), jnp.float32)], axis=0)


def _pass1_body(ea_ref, w_ref, w1e_ref, wc_ref, cv_ref, hb_ref, st_ref):
    base = (jnp.dot(w_ref[...], wc_ref[...],
                    preferred_element_type=jnp.float32, precision=lax.Precision.HIGHEST) + cv_ref[0:1, :])
    pre = jnp.dot(ea_ref[...], w1e_ref[...],
                  preferred_element_type=jnp.float32, precision=lax.Precision.HIGHEST) + base
    h = _leaky(pre)
    hb_ref[...] = jnp.concatenate([h, base], axis=1)
    _accum_stats(st_ref, h)


def _pass1(ea, w, w1e, wc, cv8):
    eblk = pl.BlockSpec((_TILE, _DE), lambda i: (i, 0))
    full = lambda shape: pl.BlockSpec(shape, lambda i: (0, 0))
    return pl.pallas_call(
        _pass1_body,
        grid=(_GRID,),
        in_specs=[eblk, eblk, full((_DE, _H)), full((_DE, _H)),
                  full((8, _H))],
        out_specs=[eblk, full((8, _H))],
        out_shape=[
            jax.ShapeDtypeStruct((_E, 2 * _H), jnp.float32),
            jax.ShapeDtypeStruct((8, _H), jnp.float32),
        ],
    )(ea, w, w1e, wc, cv8)


def _mid_body(hb_ref, g_ref, dv_ref, o_ref, st_ref):
    hb = hb_ref[...]
    pre = (jnp.dot(hb, g_ref[...], preferred_element_type=jnp.float32, precision=lax.Precision.HIGHEST)
           + dv_ref[0:1, :])
    h = _leaky(pre)
    o_ref[...] = jnp.concatenate([h, hb[:, _H:]], axis=1)
    _accum_stats(st_ref, h)


def _mid(hb, g2, dv8):
    eblk = pl.BlockSpec((_TILE, _DE), lambda i: (i, 0))
    full = lambda shape: pl.BlockSpec(shape, lambda i: (0, 0))
    return pl.pallas_call(
        _mid_body,
        grid=(_GRID,),
        in_specs=[eblk, full((_DE, _H)), full((8, _H))],
        out_specs=[eblk, full((8, _H))],
        out_shape=[
            jax.ShapeDtypeStruct((_E, 2 * _H), jnp.float32),
            jax.ShapeDtypeStruct((8, _H), jnp.float32),
        ],
    )(hb, g2, dv8)


def _final_body(hb_ref, w_ref, b_ref, o_ref):
    o_ref[...] = (jnp.dot(hb_ref[...], w_ref[...],
                          preferred_element_type=jnp.float32, precision=lax.Precision.HIGHEST) + b_ref[0:1, :])


def _final(hb, w2, b8):
    eblk = pl.BlockSpec((_TILE, _DE), lambda i: (i, 0))
    full = lambda shape: pl.BlockSpec(shape, lambda i: (0, 0))
    return pl.pallas_call(
        _final_body,
        grid=(_GRID,),
        in_specs=[eblk, full((_DE, _DE)), full((8, _DE))],
        out_specs=eblk,
        out_shape=jax.ShapeDtypeStruct((_E, _DE), jnp.float32),
    )(hb, w2, b8)


def kernel(x, edge_index, edge_attr, u, batch,
           node_W1, node_b1, node_g1, node_be1, node_W2, node_b2,
           all_W1, all_b1, all_g1, all_be1, all_W2, all_b2):
    del batch
    row = edge_index[0]
    col = edge_index[1]

    y = _node_y(x, node_W1, _pad8(node_b1))
    w, st_parts = _sc_gather(y, row, col)

    st = jnp.sum(st_parts, axis=0)
    inv_e = 1.0 / _E
    mu_r = st[0] * inv_e
    var_r = st[1] * inv_e - mu_r * mu_r
    mu_c = st[2] * inv_e
    var_c = st[3] * inv_e - mu_c * mu_c
    s_r = node_g1 * lax.rsqrt(var_r + _EPS)
    t_r = node_be1 - mu_r * s_r
    s_c = node_g1 * lax.rsqrt(var_c + _EPS)
    t_c = node_be1 - mu_c * s_c

    w1_e = all_W1[:_DE]
    w1_m = all_W1[_DE:2 * _DE]
    w1_u = all_W1[2 * _DE:]
    cmat = _hdot(node_W2, w1_m)
    wc = jnp.concatenate([s_r[:, None] * cmat, s_c[:, None] * cmat], axis=0)
    cvec = (_hdot(_hdot(t_r + t_c, node_W2) + 2.0 * node_b2, w1_m)
            + _hdot(u[0], w1_u) + all_b1)

    hb, stats = _pass1(edge_attr, w, w1_e, wc, _pad8(cvec))

    kmat = _hdot(all_W2, w1_e)
    eye = jnp.eye(_H, dtype=jnp.float32)
    for _ in range(2):
        mu = stats[0] * inv_e
        var = stats[1] * inv_e - mu * mu
        scale = all_g1 * lax.rsqrt(var + _EPS)
        shift = all_be1 - mu * scale
        g2 = jnp.concatenate([scale[:, None] * kmat, eye], axis=0)
        dvec = _hdot(_hdot(shift, all_W2) + all_b2, w1_e)
        hb, stats = _mid(hb, g2, _pad8(dvec))

    mu = stats[0] * inv_e
    var = stats[1] * inv_e - mu * mu
    scale = all_g1 * lax.rsqrt(var + _EPS)
    shift = all_be1 - mu * scale
    w2f = jnp.concatenate(
        [scale[:, None] * all_W2, jnp.zeros((_H, _DE), jnp.float32)], axis=0)
    bf = _hdot(shift, all_W2) + all_b2
    return _final(hb, w2f, _pad8(bf))


# speedup vs baseline: 1.3498x; 1.3498x over previous
import jax
import jax.numpy as jnp
from jax import lax
from jax.experimental import pallas as pl
from jax.experimental.pallas import tpu as pltpu
from jax.experimental.pallas import tpu_sc as plsc

_NEG = 0.01
_EPS = 1e-5
_N = 10000
_E = 320000
_DE = 128
_H = 64

_NW = 32
_EPW = _E // _NW
_CH = 80
_NCH = _EPW // _CH

_TILE = 2560
_GRID = _E // _TILE


def _hdot(a, b):
    return jnp.dot(a, b, preferred_element_type=jnp.float32)


def _leaky(h):
    return jnp.where(h > 0, h, _NEG * h)


def _pad8(v):
    return jnp.broadcast_to(v[None, :], (8, v.shape[0]))


def _sc_gather_body(y_hbm, row_hbm, col_hbm, w_hbm, st_hbm,
                    idx_r, idx_c, bufr, bufc, stbuf,
                    sem_r, sem_c, sem_w):
    wid = lax.axis_index("s") * 2 + lax.axis_index("c")
    ebase = wid * _EPW

    def issue(k, b):
        eoff = ebase + k * _CH
        pltpu.sync_copy(row_hbm.at[pl.ds(eoff, _CH)], idx_r.at[b])
        pltpu.sync_copy(col_hbm.at[pl.ds(eoff, _CH)], idx_c.at[b])
        pltpu.async_copy(y_hbm.at[idx_r.at[b]], bufr.at[b], sem_r)
        pltpu.async_copy(y_hbm.at[idx_c.at[b]], bufc.at[b], sem_c)

    def drain(b):
        pltpu.make_async_copy(y_hbm.at[idx_r.at[b]], bufr.at[b], sem_r).wait()
        pltpu.make_async_copy(y_hbm.at[idx_c.at[b]], bufc.at[b], sem_c).wait()

    def accum(b, acc):
        for i in range(_CH):
            for f in range(4):
                vr = bufr[b, i, pl.ds(f * 16, 16)]
                vc = bufc[b, i, pl.ds(f * 16, 16)]
                bufr[b, i, pl.ds(_H + f * 16, 16)] = vc
                acc[f] = acc[f] + vr
                acc[4 + f] = acc[4 + f] + vr * vr
                acc[8 + f] = acc[8 + f] + vc
                acc[12 + f] = acc[12 + f] + vc * vc
        return acc

    issue(0, 0)

    def chunk2(j, carry):
        acc = list(carry)
        for b in range(2):
            k = 2 * j + b
            drain(b)
            @pl.when(k > 0)
            def _():
                pltpu.make_async_copy(
                    bufr.at[1 - b], w_hbm.at[pl.ds(0, _CH)], sem_w).wait()
            issue(k + 1, 1 - b)
            acc = accum(b, acc)
            pltpu.async_copy(
                bufr.at[b], w_hbm.at[pl.ds(ebase + k * _CH, _CH)], sem_w)
        return tuple(acc)

    zero = jnp.zeros((16,), jnp.float32)
    acc = lax.fori_loop(0, _NCH // 2, chunk2, tuple(zero for _ in range(16)))
    k = _NCH - 1
    drain(0)
    pltpu.make_async_copy(
        bufr.at[1], w_hbm.at[pl.ds(0, _CH)], sem_w).wait()
    acc = accum(0, list(acc))
    pltpu.async_copy(bufr.at[0], w_hbm.at[pl.ds(ebase + k * _CH, _CH)], sem_w)
    pltpu.make_async_copy(bufr.at[0], w_hbm.at[pl.ds(0, _CH)], sem_w).wait()
    for r in range(4):
        for f in range(4):
            stbuf[r, pl.ds(f * 16, 16)] = acc[r * 4 + f]
    pltpu.sync_copy(stbuf, st_hbm.at[wid])


def _sc_gather(y, row, col):
    fn = pl.kernel(
        _sc_gather_body,
        out_type=[
            jax.ShapeDtypeStruct((_E, 2 * _H), jnp.float32),
            jax.ShapeDtypeStruct((_NW, 4, _H), jnp.float32),
        ],
        scratch_types=[
            pltpu.VMEM((2, _CH), jnp.int32),
            pltpu.VMEM((2, _CH), jnp.int32),
            pltpu.VMEM((2, _CH, 2 * _H), jnp.float32),
            pltpu.VMEM((2, _CH, 2 * _H), jnp.float32),
            pltpu.VMEM((4, _H), jnp.float32),
            pltpu.SemaphoreType.DMA,
            pltpu.SemaphoreType.DMA,
            pltpu.SemaphoreType.DMA,
        ],
        mesh=plsc.VectorSubcoreMesh(core_axis_name="c", subcore_axis_name="s"),
    )
    return fn(y, row, col)


def _node_y_body(x_ref, w_ref, b_ref, o_ref):
    h = jnp.dot(x_ref[...], w_ref[...], preferred_element_type=jnp.float32)
    y = _leaky(h + b_ref[0:1, :])
    o_ref[...] = jnp.concatenate(
        [y, jnp.zeros((_N, _H), jnp.float32)], axis=1)


def _node_y(x, w, b8):
    return pl.pallas_call(
        _node_y_body,
        out_shape=jax.ShapeDtypeStruct((_N, 2 * _H), jnp.float32),
    )(x, w, b8)


def _accum_stats(st_ref, h):
    @pl.when(pl.program_id(0) == 0)
    def _():
        st_ref[...] = jnp.zeros_like(st_ref)

    s = jnp.sum(h, axis=0, keepdims=True)
    q = jnp.sum(h * h, axis=0, keepdims=True)
    st_ref[...] += jnp.concatenate(
        [s, q, jnp.zeros((6, _H), jnp.float32)], axis=0)


def _pass1_body(ea_ref, w_ref, w1e_ref, wc_ref, cv_ref, hb_ref, st_ref):
    base = (jnp.dot(w_ref[...], wc_ref[...],
                    preferred_element_type=jnp.float32) + cv_ref[0:1, :])
    pre = jnp.dot(ea_ref[...], w1e_ref[...],
                  preferred_element_type=jnp.float32) + base
    h = _leaky(pre)
    hb_ref[...] = jnp.concatenate([h, base], axis=1).astype(jnp.bfloat16)
    _accum_stats(st_ref, h)


def _pass1(ea, w, w1e, wc, cv8):
    eblk = pl.BlockSpec((_TILE, _DE), lambda i: (i, 0))
    full = lambda shape: pl.BlockSpec(shape, lambda i: (0, 0))
    return pl.pallas_call(
        _pass1_body,
        grid=(_GRID,),
        in_specs=[eblk, eblk, full((_DE, _H)), full((_DE, _H)),
                  full((8, _H))],
        out_specs=[eblk, full((8, _H))],
        out_shape=[
            jax.ShapeDtypeStruct((_E, 2 * _H), jnp.bfloat16),
            jax.ShapeDtypeStruct((8, _H), jnp.float32),
        ],
    )(ea, w, w1e, wc, cv8)


def _mid_body(hb_ref, g_ref, dv_ref, o_ref, st_ref):
    hb = hb_ref[...].astype(jnp.float32)
    pre = (jnp.dot(hb, g_ref[...], preferred_element_type=jnp.float32)
           + dv_ref[0:1, :])
    h = _leaky(pre)
    o_ref[...] = jnp.concatenate(
        [h.astype(jnp.bfloat16), hb_ref[:, _H:]], axis=1)
    _accum_stats(st_ref, h)


def _mid(hb, g2, dv8):
    eblk = pl.BlockSpec((_TILE, _DE), lambda i: (i, 0))
    full = lambda shape: pl.BlockSpec(shape, lambda i: (0, 0))
    return pl.pallas_call(
        _mid_body,
        grid=(_GRID,),
        in_specs=[eblk, full((_DE, _H)), full((8, _H))],
        out_specs=[eblk, full((8, _H))],
        out_shape=[
            jax.ShapeDtypeStruct((_E, 2 * _H), jnp.bfloat16),
            jax.ShapeDtypeStruct((8, _H), jnp.float32),
        ],
    )(hb, g2, dv8)


def _final_body(hb_ref, w_ref, b_ref, o_ref):
    hb = hb_ref[...].astype(jnp.float32)
    o_ref[...] = (jnp.dot(hb, w_ref[...],
                          preferred_element_type=jnp.float32) + b_ref[0:1, :])


def _final(hb, w2, b8):
    eblk = pl.BlockSpec((_TILE, _DE), lambda i: (i, 0))
    full = lambda shape: pl.BlockSpec(shape, lambda i: (0, 0))
    return pl.pallas_call(
        _final_body,
        grid=(_GRID,),
        in_specs=[eblk, full((_DE, _DE)), full((8, _DE))],
        out_specs=eblk,
        out_shape=jax.ShapeDtypeStruct((_E, _DE), jnp.float32),
    )(hb, w2, b8)


def kernel(x, edge_index, edge_attr, u, batch,
           node_W1, node_b1, node_g1, node_be1, node_W2, node_b2,
           all_W1, all_b1, all_g1, all_be1, all_W2, all_b2):
    del batch
    row = edge_index[0]
    col = edge_index[1]

    y = _node_y(x, node_W1, _pad8(node_b1))
    w, st_parts = _sc_gather(y, row, col)

    st = jnp.sum(st_parts, axis=0)
    inv_e = 1.0 / _E
    mu_r = st[0] * inv_e
    var_r = st[1] * inv_e - mu_r * mu_r
    mu_c = st[2] * inv_e
    var_c = st[3] * inv_e - mu_c * mu_c
    s_r = node_g1 * lax.rsqrt(var_r + _EPS)
    t_r = node_be1 - mu_r * s_r
    s_c = node_g1 * lax.rsqrt(var_c + _EPS)
    t_c = node_be1 - mu_c * s_c

    w1_e = all_W1[:_DE]
    w1_m = all_W1[_DE:2 * _DE]
    w1_u = all_W1[2 * _DE:]
    cmat = _hdot(node_W2, w1_m)
    wc = jnp.concatenate([s_r[:, None] * cmat, s_c[:, None] * cmat], axis=0)
    cvec = (_hdot(_hdot(t_r + t_c, node_W2) + 2.0 * node_b2, w1_m)
            + _hdot(u[0], w1_u) + all_b1)

    hb, stats = _pass1(edge_attr, w, w1_e, wc, _pad8(cvec))

    kmat = _hdot(all_W2, w1_e)
    eye = jnp.eye(_H, dtype=jnp.float32)
    for _ in range(2):
        mu = stats[0] * inv_e
        var = stats[1] * inv_e - mu * mu
        scale = all_g1 * lax.rsqrt(var + _EPS)
        shift = all_be1 - mu * scale
        g2 = jnp.concatenate([scale[:, None] * kmat, eye], axis=0)
        dvec = _hdot(_hdot(shift, all_W2) + all_b2, w1_e)
        hb, stats = _mid(hb, g2, _pad8(dvec))

    mu = stats[0] * inv_e
    var = stats[1] * inv_e - mu * mu
    scale = all_g1 * lax.rsqrt(var + _EPS)
    shift = all_be1 - mu * scale
    w2f = jnp.concatenate(
        [scale[:, None] * all_W2, jnp.zeros((_H, _DE), jnp.float32)], axis=0)
    bf = _hdot(shift, all_W2) + all_b2
    return _final(hb, w2f, _pad8(bf))


# speedup vs baseline: 1.3776x; 1.0205x over previous
import jax
import jax.numpy as jnp
from jax import lax
from jax.experimental import pallas as pl
from jax.experimental.pallas import tpu as pltpu
from jax.experimental.pallas import tpu_sc as plsc

_NEG = 0.01
_EPS = 1e-5
_N = 10000
_E = 320000
_DE = 128
_H = 64

_NW = 32
_EPW = _E // _NW
_CH = 80
_NCH = _EPW // _CH

_TILE = 2560
_GRID = _E // _TILE


def _hdot(a, b):
    return jnp.dot(a, b, preferred_element_type=jnp.float32)


def _leaky(h):
    return jnp.where(h > 0, h, _NEG * h)


def _pad8(v):
    return jnp.broadcast_to(v[None, :], (8, v.shape[0]))


def _sc_gather_body(y_hbm, row_hbm, col_hbm, w_hbm,
                    idx_r, idx_c, bufr, bufc,
                    sem_r, sem_c, sem_w):
    wid = lax.axis_index("s") * 2 + lax.axis_index("c")
    ebase = wid * _EPW

    def issue(k, b):
        eoff = ebase + k * _CH
        pltpu.sync_copy(row_hbm.at[pl.ds(eoff, _CH)], idx_r.at[b])
        pltpu.sync_copy(col_hbm.at[pl.ds(eoff, _CH)], idx_c.at[b])
        pltpu.async_copy(y_hbm.at[idx_r.at[b]], bufr.at[b], sem_r)
        pltpu.async_copy(y_hbm.at[idx_c.at[b]], bufc.at[b], sem_c)

    def drain(b):
        pltpu.make_async_copy(y_hbm.at[idx_r.at[b]], bufr.at[b], sem_r).wait()
        pltpu.make_async_copy(y_hbm.at[idx_c.at[b]], bufc.at[b], sem_c).wait()

    def merge(b):
        for i in range(_CH):
            for f in range(4):
                bufr[b, i, pl.ds(_H + f * 16, 16)] = bufc[b, i, pl.ds(f * 16, 16)]

    issue(0, 0)

    def chunk2(j, carry):
        for b in range(2):
            k = 2 * j + b
            drain(b)
            @pl.when(k > 0)
            def _():
                pltpu.make_async_copy(
                    bufr.at[1 - b], w_hbm.at[pl.ds(0, _CH)], sem_w).wait()

            @pl.when(k + 1 < _NCH)
            def _():
                issue(k + 1, 1 - b)
            merge(b)
            pltpu.async_copy(
                bufr.at[b], w_hbm.at[pl.ds(ebase + k * _CH, _CH)], sem_w)
        return carry

    lax.fori_loop(0, _NCH // 2, chunk2, 0)
    k = _NCH - 1
    drain(0)
    pltpu.make_async_copy(
        bufr.at[1], w_hbm.at[pl.ds(0, _CH)], sem_w).wait()
    merge(0)
    pltpu.async_copy(bufr.at[0], w_hbm.at[pl.ds(ebase + k * _CH, _CH)], sem_w)
    pltpu.make_async_copy(bufr.at[0], w_hbm.at[pl.ds(0, _CH)], sem_w).wait()


def _sc_gather(y, row, col):
    fn = pl.kernel(
        _sc_gather_body,
        out_type=jax.ShapeDtypeStruct((_E, 2 * _H), jnp.float32),
        scratch_types=[
            pltpu.VMEM((2, _CH), jnp.int32),
            pltpu.VMEM((2, _CH), jnp.int32),
            pltpu.VMEM((2, _CH, 2 * _H), jnp.float32),
            pltpu.VMEM((2, _CH, 2 * _H), jnp.float32),
            pltpu.SemaphoreType.DMA,
            pltpu.SemaphoreType.DMA,
            pltpu.SemaphoreType.DMA,
        ],
        mesh=plsc.VectorSubcoreMesh(core_axis_name="c", subcore_axis_name="s"),
    )
    return fn(y, row, col)


def _node_y_body(x_ref, w_ref, b_ref, o_ref):
    h = jnp.dot(x_ref[...], w_ref[...], preferred_element_type=jnp.float32)
    y = _leaky(h + b_ref[0:1, :])
    o_ref[...] = jnp.concatenate(
        [y, jnp.zeros((_N, _H), jnp.float32)], axis=1)


def _node_y(x, w, b8):
    return pl.pallas_call(
        _node_y_body,
        out_shape=jax.ShapeDtypeStruct((_N, 2 * _H), jnp.float32),
    )(x, w, b8)


def _wstats_body(w_ref, st_ref):
    @pl.when(pl.program_id(0) == 0)
    def _():
        st_ref[...] = jnp.zeros_like(st_ref)

    w = w_ref[...]
    s = jnp.sum(w, axis=0, keepdims=True)
    q = jnp.sum(w * w, axis=0, keepdims=True)
    st_ref[...] += jnp.concatenate(
        [s, q, jnp.zeros((6, _DE), jnp.float32)], axis=0)


def _wstats(w):
    return pl.pallas_call(
        _wstats_body,
        grid=(_GRID,),
        in_specs=[pl.BlockSpec((_TILE, _DE), lambda i: (i, 0))],
        out_specs=pl.BlockSpec((8, _DE), lambda i: (0, 0)),
        out_shape=jax.ShapeDtypeStruct((8, _DE), jnp.float32),
    )(w)


def _accum_stats(st_ref, h):
    @pl.when(pl.program_id(0) == 0)
    def _():
        st_ref[...] = jnp.zeros_like(st_ref)

    s = jnp.sum(h, axis=0, keepdims=True)
    q = jnp.sum(h * h, axis=0, keepdims=True)
    st_ref[...] += jnp.concatenate(
        [s, q, jnp.zeros((6, _H), jnp.float32)], axis=0)


def _pass1_body(ea_ref, w_ref, w1e_ref, wc_ref, cv_ref, hb_ref, st_ref):
    base = (jnp.dot(w_ref[...], wc_ref[...],
                    preferred_element_type=jnp.float32) + cv_ref[0:1, :])
    pre = jnp.dot(ea_ref[...], w1e_ref[...],
                  preferred_element_type=jnp.float32) + base
    h = _leaky(pre)
    hb_ref[...] = jnp.concatenate([h, base], axis=1).astype(jnp.bfloat16)
    _accum_stats(st_ref, h)


def _pass1(ea, w, w1e, wc, cv8):
    eblk = pl.BlockSpec((_TILE, _DE), lambda i: (i, 0))
    full = lambda shape: pl.BlockSpec(shape, lambda i: (0, 0))
    return pl.pallas_call(
        _pass1_body,
        grid=(_GRID,),
        in_specs=[eblk, eblk, full((_DE, _H)), full((_DE, _H)),
                  full((8, _H))],
        out_specs=[eblk, full((8, _H))],
        out_shape=[
            jax.ShapeDtypeStruct((_E, 2 * _H), jnp.bfloat16),
            jax.ShapeDtypeStruct((8, _H), jnp.float32),
        ],
    )(ea, w, w1e, wc, cv8)


def _mid_body(hb_ref, g_ref, dv_ref, o_ref, st_ref):
    hb = hb_ref[...].astype(jnp.float32)
    pre = (jnp.dot(hb, g_ref[...], preferred_element_type=jnp.float32)
           + dv_ref[0:1, :])
    h = _leaky(pre)
    o_ref[...] = jnp.concatenate(
        [h.astype(jnp.bfloat16), hb_ref[:, _H:]], axis=1)
    _accum_stats(st_ref, h)


def _mid(hb, g2, dv8):
    eblk = pl.BlockSpec((_TILE, _DE), lambda i: (i, 0))
    full = lambda shape: pl.BlockSpec(shape, lambda i: (0, 0))
    return pl.pallas_call(
        _mid_body,
        grid=(_GRID,),
        in_specs=[eblk, full((_DE, _H)), full((8, _H))],
        out_specs=[eblk, full((8, _H))],
        out_shape=[
            jax.ShapeDtypeStruct((_E, 2 * _H), jnp.bfloat16),
            jax.ShapeDtypeStruct((8, _H), jnp.float32),
        ],
    )(hb, g2, dv8)


def _final_body(hb_ref, w_ref, b_ref, o_ref):
    hb = hb_ref[...].astype(jnp.float32)
    o_ref[...] = (jnp.dot(hb, w_ref[...],
                          preferred_element_type=jnp.float32) + b_ref[0:1, :])


def _final(hb, w2, b8):
    eblk = pl.BlockSpec((_TILE, _DE), lambda i: (i, 0))
    full = lambda shape: pl.BlockSpec(shape, lambda i: (0, 0))
    return pl.pallas_call(
        _final_body,
        grid=(_GRID,),
        in_specs=[eblk, full((_DE, _DE)), full((8, _DE))],
        out_specs=eblk,
        out_shape=jax.ShapeDtypeStruct((_E, _DE), jnp.float32),
    )(hb, w2, b8)


def kernel(x, edge_index, edge_attr, u, batch,
           node_W1, node_b1, node_g1, node_be1, node_W2, node_b2,
           all_W1, all_b1, all_g1, all_be1, all_W2, all_b2):
    del batch
    row = edge_index[0]
    col = edge_index[1]

    y = _node_y(x, node_W1, _pad8(node_b1))
    w = _sc_gather(y, row, col)

    wst = _wstats(w)
    inv_e = 1.0 / _E
    mu_r = wst[0, :_H] * inv_e
    var_r = wst[1, :_H] * inv_e - mu_r * mu_r
    mu_c = wst[0, _H:] * inv_e
    var_c = wst[1, _H:] * inv_e - mu_c * mu_c
    s_r = node_g1 * lax.rsqrt(var_r + _EPS)
    t_r = node_be1 - mu_r * s_r
    s_c = node_g1 * lax.rsqrt(var_c + _EPS)
    t_c = node_be1 - mu_c * s_c

    w1_e = all_W1[:_DE]
    w1_m = all_W1[_DE:2 * _DE]
    w1_u = all_W1[2 * _DE:]
    cmat = _hdot(node_W2, w1_m)
    wc = jnp.concatenate([s_r[:, None] * cmat, s_c[:, None] * cmat], axis=0)
    cvec = (_hdot(_hdot(t_r + t_c, node_W2) + 2.0 * node_b2, w1_m)
            + _hdot(u[0], w1_u) + all_b1)

    hb, stats = _pass1(edge_attr, w, w1_e, wc, _pad8(cvec))

    kmat = _hdot(all_W2, w1_e)
    eye = jnp.eye(_H, dtype=jnp.float32)
    for _ in range(2):
        mu = stats[0] * inv_e
        var = stats[1] * inv_e - mu * mu
        scale = all_g1 * lax.rsqrt(var + _EPS)
        shift = all_be1 - mu * scale
        g2 = jnp.concatenate([scale[:, None] * kmat, eye], axis=0)
        dvec = _hdot(_hdot(shift, all_W2) + all_b2, w1_e)
        hb, stats = _mid(hb, g2, _pad8(dvec))

    mu = stats[0] * inv_e
    var = stats[1] * inv_e - mu * mu
    scale = all_g1 * lax.rsqrt(var + _EPS)
    shift = all_be1 - mu * scale
    w2f = jnp.concatenate(
        [scale[:, None] * all_W2, jnp.zeros((_H, _DE), jnp.float32)], axis=0)
    bf = _hdot(shift, all_W2) + all_b2
    return _final(hb, w2f, _pad8(bf))
